# baseline (speedup 1.0000x reference)
import functools

import jax
import jax.numpy as jnp
from jax.experimental import pallas as pl

B = 16384
INPUT_DIM = 512
EMB_DIM = 128
N_CLUSTERS = 64
BLOCK_ROWS = 1024


def _fused_body(x_ref, pca_ref, cent_ref, out_ref):
    xb = x_ref[...]
    pe = pca_ref[...]
    cen = cent_ref[...]

    x_enc = jax.lax.dot_general(
        xb, pe, (((1,), (1,)), ((), ())),
        preferred_element_type=jnp.float32)

    cross = jax.lax.dot_general(
        x_enc, cen, (((1,), (1,)), ((), ())),
        preferred_element_type=jnp.float32)

    x2 = jnp.sum(x_enc * x_enc, axis=1, keepdims=True)
    c2 = jnp.sum(cen * cen, axis=1)[None, :]
    d2 = x2 + c2 - 2.0 * cross
    m = jnp.min(d2, axis=1)
    out_ref[...] = jnp.sqrt(jnp.maximum(m, 0.0))


@functools.partial(jax.jit, static_argnames=("interpret",))
def kernel(x, pca_components, centroids, interpret=False):
    grid = (B // BLOCK_ROWS,)
    return pl.pallas_call(
        _fused_body,
        grid=grid,
        in_specs=[
            pl.BlockSpec((BLOCK_ROWS, INPUT_DIM), lambda i: (i, 0)),
            pl.BlockSpec((EMB_DIM, INPUT_DIM), lambda i: (0, 0)),
            pl.BlockSpec((N_CLUSTERS, EMB_DIM), lambda i: (0, 0)),
        ],
        out_specs=pl.BlockSpec((BLOCK_ROWS,), lambda i: (i,)),
        out_shape=jax.ShapeDtypeStruct((B,), jnp.float32),
        interpret=interpret,
    )(x, pca_components, centroids)
